# Initial kernel scaffold; baseline (speedup 1.0000x reference)
import jax
import jax.numpy as jnp
from jax.experimental import pallas as pl


def kernel(rgb, flow, rgb_log, flow_log, count, head):
    raise NotImplementedError("write your pallas kernel here")


# speedup vs baseline: 4.3698x; 4.3698x over previous
import functools

import jax
import jax.numpy as jnp
from jax import lax
from jax.experimental import pallas as pl
from jax.experimental.pallas import tpu as pltpu
from jax.experimental.pallas import tpu_sc as plsc

C = 101
D = 1024
M = 512
TOPK = 5
PAD = 8

NC = 2
NS = 16
NW = NC * NS
LANES = 16


def _select_kernel(cnt_ref, rl_ref, fl_ref, gidx_ref, w_ref):
    c = pl.program_id(0)
    n = cnt_ref[c]

    def _avg_entropy(ref):
        x = ref[0]
        m = jnp.max(x, axis=-1, keepdims=True)
        e = jnp.exp(x - m)
        z = jnp.sum(e, axis=-1, keepdims=True)
        p = e / z
        return -jnp.sum(p * jnp.log(p + 1e-08), axis=-1, keepdims=True)

    avg_h = (_avg_entropy(rl_ref) + _avg_entropy(fl_ref)) * 0.5

    row = lax.broadcasted_iota(jnp.int32, (M, 1), 0)
    masked = jnp.where(row < n, avg_h, jnp.inf)

    ids = []
    for _ in range(TOPK):
        mv = jnp.min(masked)
        im = jnp.min(jnp.where(masked == mv, row, M))
        ids.append(im)
        masked = jnp.where(row == im, jnp.inf, masked)

    j8 = lax.broadcasted_iota(jnp.int32, (1, 1, PAD), 2)
    top = jnp.zeros((1, 1, PAD), jnp.int32)
    for j, s in enumerate(ids):
        top = jnp.where(j8 == j, s, top)

    use_top = n > TOPK
    local = jnp.where(use_top, top, j8)
    nf = jnp.maximum(n, 1).astype(jnp.float32)
    jl = lax.broadcasted_iota(jnp.int32, (1, 1, PAD * LANES), 2) // LANES
    w_mean = jnp.where(jl < n, 1.0 / nf, 0.0)
    w_top = jnp.where(jl < TOPK, 1.0 / TOPK, 0.0)

    gidx_ref[...] = c * M + local
    w_ref[...] = jnp.where(use_top, w_top, w_mean)


def _select(rgb_log, flow_log, count):
    grid_spec = pltpu.PrefetchScalarGridSpec(
        num_scalar_prefetch=1,
        grid=(C,),
        in_specs=[
            pl.BlockSpec((1, M, C), lambda c, cnt: (c, 0, 0)),
            pl.BlockSpec((1, M, C), lambda c, cnt: (c, 0, 0)),
        ],
        out_specs=[
            pl.BlockSpec((1, 1, PAD), lambda c, cnt: (c, 0, 0)),
            pl.BlockSpec((1, 1, PAD * LANES), lambda c, cnt: (c, 0, 0)),
        ],
    )
    gidx, w = pl.pallas_call(
        _select_kernel,
        grid_spec=grid_spec,
        out_shape=[
            jax.ShapeDtypeStruct((C, 1, PAD), jnp.int32),
            jax.ShapeDtypeStruct((C, 1, PAD * LANES), jnp.float32),
        ],
    )(count, rgb_log, flow_log)
    return gidx.reshape(C, PAD), w.reshape(C, PAD * LANES)


def _gather_body(rgb_hbm, flow_hbm, gidx_hbm, w_hbm, rgb_out, flow_out,
                 idx_v, w_v, rows_r, rows_f, out_v, sem_r, sem_f):
    wid = lax.axis_index("s") * NC + lax.axis_index("c")

    def _weighted_sum(rows, out_hbm, c):
        wvec = [w_v[pl.ds(j * LANES, LANES)] for j in range(PAD)]

        def chunk(k, _):
            acc = jnp.zeros((LANES,), jnp.float32)
            for j in range(PAD):
                acc = acc + wvec[j] * rows[j, pl.ds(k * LANES, LANES)]
            out_v[pl.ds(k * LANES, LANES)] = acc
            return 0

        lax.fori_loop(0, D // LANES, chunk, 0, unroll=4)
        pltpu.sync_copy(out_v, out_hbm.at[c])

    def per_class(t, _):
        c = wid + NW * t

        @pl.when(c < C)
        def _():
            pltpu.sync_copy(gidx_hbm.at[c], idx_v)
            pltpu.sync_copy(w_hbm.at[c], w_v)
            cp_r = pltpu.async_copy(rgb_hbm.at[idx_v], rows_r, sem_r)
            cp_f = pltpu.async_copy(flow_hbm.at[idx_v], rows_f, sem_f)
            cp_r.wait()
            _weighted_sum(rows_r, rgb_out, c)
            cp_f.wait()
            _weighted_sum(rows_f, flow_out, c)

        return 0

    lax.fori_loop(0, (C + NW - 1) // NW, per_class, 0)


def _gather_mean(rgb_flat, flow_flat, gidx, w):
    mesh = plsc.VectorSubcoreMesh(core_axis_name="c", subcore_axis_name="s")
    return pl.kernel(
        _gather_body,
        out_type=[
            jax.ShapeDtypeStruct((C, D), jnp.float32),
            jax.ShapeDtypeStruct((C, D), jnp.float32),
        ],
        mesh=mesh,
        scratch_types=[
            pltpu.VMEM((PAD,), jnp.int32),
            pltpu.VMEM((PAD * LANES,), jnp.float32),
            pltpu.VMEM((PAD, D), jnp.float32),
            pltpu.VMEM((PAD, D), jnp.float32),
            pltpu.VMEM((D,), jnp.float32),
            pltpu.SemaphoreType.DMA,
            pltpu.SemaphoreType.DMA,
        ],
    )(rgb_flat, flow_flat, gidx, w)


def kernel(rgb, flow, rgb_log, flow_log, count, head):
    del head
    count = count.astype(jnp.int32)
    gidx, w = _select(rgb_log, flow_log, count)
    rgb_p, flow_p = _gather_mean(
        rgb.reshape(C * M, D), flow.reshape(C * M, D), gidx, w)
    return (rgb_p, flow_p)
